# baseline (speedup 1.0000x reference)
import functools

import jax
import jax.numpy as jnp
from jax.experimental import pallas as pl
from jax.experimental.pallas import tpu as pltpu

N = 10000
D = 256
H = 256
K1 = 17
TEMP = 0.5
EPS = 1e-8

NP = 10240
RB = 256
CT = 2048
NCT = NP // CT
NRB = NP // RB

MANT_BITS = 14
MANT_MASK = (1 << MANT_BITS) - 1


def _ew_call(fn, out_shapes, *args):
    return pl.pallas_call(
        fn,
        out_shape=out_shapes,
    )(*args)


def _xp_kernel(x_ref, pf_ref, ps_ref, o_ref):
    o_ref[...] = jax.nn.relu(x_ref[...] + pf_ref[...]) + ps_ref[...]


def _matmul_bias_act_kernel(act, x_ref, w_ref, b_ref, o_ref):
    y = jnp.dot(x_ref[...], w_ref[...], preferred_element_type=jnp.float32)
    y = y + b_ref[...]
    if act == "relu":
        y = jax.nn.relu(y)
    elif act == "elu":
        y = jnp.where(y > 0, y, jnp.expm1(y))
    o_ref[...] = y


def _matmul_bias_act(x, w, b, act):
    npad, din = x.shape
    dout = w.shape[1]
    rb = 512
    return pl.pallas_call(
        functools.partial(_matmul_bias_act_kernel, act),
        grid=(npad // rb,),
        in_specs=[
            pl.BlockSpec((rb, din), lambda i: (i, 0)),
            pl.BlockSpec((din, dout), lambda i: (0, 0)),
            pl.BlockSpec((1, dout), lambda i: (0, 0)),
        ],
        out_specs=pl.BlockSpec((rb, dout), lambda i: (i, 0)),
        out_shape=jax.ShapeDtypeStruct((npad, dout), jnp.float32),
    )(x, w, b.reshape(1, dout))


def _rownorm_kernel(eps, x_ref, o_ref):
    x = x_ref[...]
    n = jnp.sqrt(jnp.sum(x * x, axis=1, keepdims=True))
    o_ref[...] = x / jnp.maximum(n, eps)


def _rownormalize(x, eps=EPS):
    npad, d = x.shape
    rb = 512
    return pl.pallas_call(
        functools.partial(_rownorm_kernel, eps),
        grid=(npad // rb,),
        in_specs=[pl.BlockSpec((rb, d), lambda i: (i, 0))],
        out_specs=pl.BlockSpec((rb, d), lambda i: (i, 0)),
        out_shape=jax.ShapeDtypeStruct((npad, d), jnp.float32),
    )(x)


def _sim_topk_kernel(hnb_ref, hnt_ref, vals_ref, idx_ref, strip_ref):
    j = pl.program_id(1)
    blk = jnp.dot(hnb_ref[...], hnt_ref[...],
                  preferred_element_type=jnp.float32)
    col = jax.lax.broadcasted_iota(jnp.int32, (RB, CT), 1) + j * CT
    blk = jnp.where(col < N, blk, -2.0)
    p = jax.lax.bitcast_convert_type(blk, jnp.int32)
    p = (p & jnp.int32(~MANT_MASK)) | col
    strip_ref[:, pl.ds(j * CT, CT)] = jax.lax.bitcast_convert_type(
        p, jnp.float32)

    @pl.when(j == NCT - 1)
    def _():
        s = strip_ref[...]
        vals_cols = []
        idx_cols = []
        for _ in range(K1):
            m = jnp.max(s, axis=1, keepdims=True)
            s = jnp.where(s == m, -3.0, s)
            mi = jax.lax.bitcast_convert_type(m, jnp.int32)
            vals_cols.append(jax.lax.bitcast_convert_type(
                mi & jnp.int32(~MANT_MASK), jnp.float32))
            idx_cols.append(mi & MANT_MASK)
        pad_f = jnp.zeros((RB, 128 - K1), jnp.float32)
        pad_i = jnp.zeros((RB, 128 - K1), jnp.int32)
        vals_ref[...] = jnp.concatenate(vals_cols + [pad_f], axis=1)
        idx_ref[...] = jnp.concatenate(idx_cols + [pad_i], axis=1)


def _sim_topk(hn, hnt):
    return pl.pallas_call(
        _sim_topk_kernel,
        grid=(NRB, NCT),
        in_specs=[
            pl.BlockSpec((RB, 2 * D), lambda i, j: (i, 0)),
            pl.BlockSpec((2 * D, CT), lambda i, j: (0, j)),
        ],
        out_specs=[
            pl.BlockSpec((RB, 128), lambda i, j: (i, 0)),
            pl.BlockSpec((RB, 128), lambda i, j: (i, 0)),
        ],
        out_shape=[
            jax.ShapeDtypeStruct((NP, 128), jnp.float32),
            jax.ShapeDtypeStruct((NP, 128), jnp.int32),
        ],
        scratch_shapes=[pltpu.VMEM((RB, NP), jnp.float32)],
        compiler_params=pltpu.CompilerParams(
            dimension_semantics=("arbitrary", "arbitrary")),
    )(hn, hnt)


def _lse_kernel(z1b_ref, z2t_ref, lse_ref, acc_ref):
    j = pl.program_id(1)

    @pl.when(j == 0)
    def _():
        acc_ref[...] = jnp.zeros_like(acc_ref)

    blk = jnp.dot(z1b_ref[...], z2t_ref[...],
                  preferred_element_type=jnp.float32) * (1.0 / TEMP)
    col = jax.lax.broadcasted_iota(jnp.int32, (RB, CT), 1) + j * CT
    blk = jnp.where(col < N, blk, -jnp.inf)
    acc_ref[...] += jnp.exp(blk)

    @pl.when(j == NCT - 1)
    def _():
        se = jnp.sum(acc_ref[...], axis=1, keepdims=True)
        lse_ref[...] = jnp.broadcast_to(jnp.log(se), (RB, 128))


def _lse(z1n, z2t):
    out = pl.pallas_call(
        _lse_kernel,
        grid=(NRB, NCT),
        in_specs=[
            pl.BlockSpec((RB, H), lambda i, j: (i, 0)),
            pl.BlockSpec((H, CT), lambda i, j: (0, j)),
        ],
        out_specs=pl.BlockSpec((RB, 128), lambda i, j: (i, 0)),
        out_shape=jax.ShapeDtypeStruct((NP, 128), jnp.float32),
        scratch_shapes=[pltpu.VMEM((RB, CT), jnp.float32)],
        compiler_params=pltpu.CompilerParams(
            dimension_semantics=("arbitrary", "arbitrary")),
    )(z1n, z2t)
    return out[:, 0]


def _loss_kernel(diag_ref, lse_ref, lpnf_ref, lpnb_ref, o_ref):
    rows = jax.lax.broadcasted_iota(jnp.int32, (NP // 128, 128), 0)
    cols = jax.lax.broadcasted_iota(jnp.int32, (NP // 128, 128), 1)
    valid = (rows * 128 + cols) < N

    pii = jnp.exp(diag_ref[...] - lse_ref[...])
    t_self = -jnp.log(jnp.maximum(pii, EPS))
    t_f = -jnp.log(jnp.maximum(lpnf_ref[...], EPS))
    t_b = -jnp.log(jnp.maximum(lpnb_ref[...], EPS))
    tot = t_self + 0.5 * (t_f + t_b)
    tot = jnp.where(valid, tot, 0.0)
    o_ref[0, 0] = jnp.sum(tot) * (1.0 / N)


def _final_loss(diag, lse, lpn_f, lpn_b):
    r = lambda a: a.reshape(NP // 128, 128)
    out = pl.pallas_call(
        _loss_kernel,
        out_shape=jax.ShapeDtypeStruct((1, 1), jnp.float32),
        out_specs=pl.BlockSpec(memory_space=pltpu.SMEM),
    )(r(diag), r(lse), r(lpn_f), r(lpn_b))
    return out[0, 0]


def _segment_sum(vals, ids, num):
    return jax.ops.segment_sum(vals, ids, num_segments=num)


def _symmetrize(idx, vals):
    nbr_rows = jnp.take(idx, idx.reshape(-1), axis=0)
    nbr_vals = jnp.take(vals, idx.reshape(-1), axis=0)
    me = jnp.repeat(jnp.arange(N, dtype=idx.dtype), K1)[:, None]
    match = nbr_rows == me
    rev_v = jnp.sum(jnp.where(match, nbr_vals, 0.0), axis=1)
    has = jnp.any(match, axis=1)
    s = (vals.reshape(-1) + rev_v) / (1.0 + has.astype(jnp.float32))
    s = jax.nn.relu(s)
    s2 = jnp.where(has, 0.0, s)
    return s, s2


def kernel(x, edge_index, batch, prompt_feat, prompt_shared, prompt_balance,
           W1, b1, W2, b2):
    del batch
    src = edge_index[0].astype(jnp.int32)
    dst = edge_index[1].astype(jnp.int32)

    xp = _ew_call(
        _xp_kernel, jax.ShapeDtypeStruct((N, D), jnp.float32),
        x, jnp.broadcast_to(prompt_feat, (1, D)),
        jnp.broadcast_to(prompt_shared, (1, D)))

    ones = jnp.ones_like(src, dtype=jnp.float32)
    deg = _segment_sum(ones, dst, N) + 1.0
    dinv = deg ** -0.5
    dinv2 = dinv * dinv

    def prop(y):
        ys = y * dinv[:, None]
        agg = _segment_sum(ys[src], dst, N)
        return dinv[:, None] * agg + dinv2[:, None] * y

    agg = prop(xp)
    h = jnp.concatenate([xp, agg], axis=1) + prompt_balance[None, :]

    hp = jnp.pad(h, ((0, NP - N), (0, 0)))
    hn = _rownormalize(hp)
    hn = jnp.where(jnp.arange(NP)[:, None] < N, hn, 0.0)
    hnt = hn.T

    vals_p, idx_p = _sim_topk(hn, hnt)
    vals = vals_p[:N, :K1]
    idx = idx_p[:N, :K1]

    s, s2 = _symmetrize(idx, vals)
    base = jnp.arange(N, dtype=jnp.int32)
    a_edges = jnp.concatenate([jnp.repeat(base, K1), idx.reshape(-1)])
    b_edges = jnp.concatenate([idx.reshape(-1), jnp.repeat(base, K1)])
    w_edges = jnp.concatenate([s, s2])

    xp_p = jnp.pad(xp, ((0, NP - N), (0, 0)))
    y1 = _matmul_bias_act(xp_p, W1, jnp.zeros((H,), jnp.float32),
                          "none")[:N]
    h1 = jax.nn.relu(prop(y1) + b1[None, :])
    h1_p = jnp.pad(h1, ((0, NP - N), (0, 0)))
    y2 = _matmul_bias_act(h1_p, W2, jnp.zeros((H,), jnp.float32),
                          "none")[:N]
    z1 = jax.nn.elu(prop(y2) + b2[None, :])

    def spmm_knn(y):
        contrib = w_edges[:, None] * y[a_edges]
        return _segment_sum(contrib, b_edges, N)

    h2 = jax.nn.relu(spmm_knn(y1) + b1[None, :])
    h2_p = jnp.pad(h2, ((0, NP - N), (0, 0)))
    y2b = _matmul_bias_act(h2_p, W2, jnp.zeros((H,), jnp.float32),
                           "none")[:N]
    z2 = jax.nn.elu(spmm_knn(y2b) + b2[None, :])

    z1n = _rownormalize(jnp.pad(z1, ((0, NP - N), (0, 0))))
    z2n = _rownormalize(jnp.pad(z2, ((0, NP - N), (0, 0))))
    z1n = jnp.where(jnp.arange(NP)[:, None] < N, z1n, 0.0)
    z2n = jnp.where(jnp.arange(NP)[:, None] < N, z2n, 0.0)

    diag = jnp.sum(z1n * z2n, axis=1) * (1.0 / TEMP)
    lse = _lse(z1n, z2n.T)

    simz_e = jnp.sum(z1n[a_edges] * z2n[b_edges], axis=1) * (1.0 / TEMP)
    t = w_edges * jnp.exp(simz_e - lse[a_edges])
    lpn_f = _segment_sum(t, a_edges, N)
    lpn_b = _segment_sum(t, b_edges, N)

    pad1 = jnp.ones((NP - N,), jnp.float32)
    return _final_loss(
        diag,
        lse,
        jnp.concatenate([lpn_f, pad1]),
        jnp.concatenate([lpn_b, pad1]),
    )
